# initial kernel scaffold (unmeasured)
import jax
import jax.numpy as jnp
from jax import lax
from jax.experimental import pallas as pl
from jax.experimental.pallas import tpu as pltpu


def kernel(
    x,
):
    def body(*refs):
        pass

    out_shape = jax.ShapeDtypeStruct(..., jnp.float32)
    return pl.pallas_call(body, out_shape=out_shape)(...)


# baseline (device time: 309633 ns/iter reference)
import jax
import jax.numpy as jnp
from jax import lax
from jax.experimental import pallas as pl
from jax.experimental.pallas import tpu as pltpu

N_DEV = 32
M = 2048
N = 1024
ROWS = M // N_DEV


def kernel(x):
    def body(x_ref, out_ref, comm_ref, send_sem, rs_recv_sems, ag_recv_sems):
        my = lax.axis_index("i")
        right = lax.rem(my + 1, N_DEV)

        out_ref[:, :] = x_ref[0, :, :]

        for s in range(N_DEV - 1):
            c_send = lax.rem(my - s + N_DEV, N_DEV)
            rdma = pltpu.make_async_remote_copy(
                src_ref=out_ref.at[pl.ds(c_send * ROWS, ROWS), :],
                dst_ref=comm_ref.at[s],
                send_sem=send_sem,
                recv_sem=rs_recv_sems.at[s],
                device_id=(right,),
                device_id_type=pl.DeviceIdType.MESH,
            )
            rdma.start()
            rdma.wait()
            c_recv = lax.rem(my - s - 1 + N_DEV, N_DEV)
            off = c_recv * ROWS
            out_ref[pl.ds(off, ROWS), :] = (
                out_ref[pl.ds(off, ROWS), :] + comm_ref[s, :, :]
            )

        for t in range(N_DEV - 1):
            c = lax.rem(my + 1 - t + N_DEV, N_DEV)
            off = c * ROWS
            rdma = pltpu.make_async_remote_copy(
                src_ref=out_ref.at[pl.ds(off, ROWS), :],
                dst_ref=out_ref.at[pl.ds(off, ROWS), :],
                send_sem=send_sem,
                recv_sem=ag_recv_sems.at[t],
                device_id=(right,),
                device_id_type=pl.DeviceIdType.MESH,
            )
            rdma.start()
            rdma.wait()

    return pl.pallas_call(
        body,
        out_shape=jax.ShapeDtypeStruct((M, N), jnp.float32),
        in_specs=[pl.BlockSpec(memory_space=pltpu.VMEM)],
        out_specs=pl.BlockSpec(memory_space=pltpu.VMEM),
        scratch_shapes=[
            pltpu.VMEM((N_DEV - 1, ROWS, N), jnp.float32),
            pltpu.SemaphoreType.DMA,
            pltpu.SemaphoreType.DMA((N_DEV - 1,)),
            pltpu.SemaphoreType.DMA((N_DEV - 1,)),
        ],
    )(x)


# device time: 221336 ns/iter; 1.3989x vs baseline; 1.3989x over previous
import jax
import jax.numpy as jnp
from jax import lax
from jax.experimental import pallas as pl
from jax.experimental.pallas import tpu as pltpu

M = 2048
N = 1024
HALF = M // 2
QTR = HALF // 4
SUB = QTR // 4


def kernel(x):
    def body(x_ref, out_ref, xbuf, ybuf, zbuf, send_sem, recv_sems):
        p = lax.axis_index("i")
        z = p // 8
        r = lax.rem(p, 8)
        y = r // 2
        q = lax.rem(r, 2)
        xc = jnp.where(lax.rem(y, 2) == 0, q, 1 - q)

        def pos(xx, yy, zz):
            return zz * 8 + yy * 2 + jnp.where(lax.rem(yy, 2) == 0, xx, 1 - xx)

        x_partner = pos(1 - xc, y, z)
        y_right = pos(xc, lax.rem(y + 1, 4), z)
        z_right = pos(xc, y, lax.rem(z + 1, 4))

        base = xc * HALF
        base2 = base + y * QTR

        out_ref[:, :] = x_ref[0, :, :]

        sem_i = 0

        def step(src_slc, dst_ref, dev, acc=None):
            nonlocal sem_i
            rdma = pltpu.make_async_remote_copy(
                src_ref=src_slc,
                dst_ref=dst_ref,
                send_sem=send_sem,
                recv_sem=recv_sems.at[sem_i],
                device_id=(dev,),
                device_id_type=pl.DeviceIdType.MESH,
            )
            rdma.start()
            rdma.wait()
            sem_i += 1
            if acc is not None:
                off, rows, load = acc
                out_ref[pl.ds(off, rows), :] = out_ref[pl.ds(off, rows), :] + load()

        step(
            out_ref.at[pl.ds((1 - xc) * HALF, HALF), :],
            xbuf,
            x_partner,
            acc=(base, HALF, lambda: xbuf[:, :]),
        )

        for s in range(3):
            c_send = lax.rem(y - s - 1 + 8, 4)
            c_recv = lax.rem(y - s - 2 + 8, 4)
            step(
                out_ref.at[pl.ds(base + c_send * QTR, QTR), :],
                ybuf.at[s],
                y_right,
                acc=(base + c_recv * QTR, QTR, lambda s=s: ybuf[s, :, :]),
            )

        for s in range(3):
            d_send = lax.rem(z - s - 1 + 8, 4)
            d_recv = lax.rem(z - s - 2 + 8, 4)
            step(
                out_ref.at[pl.ds(base2 + d_send * SUB, SUB), :],
                zbuf.at[s],
                z_right,
                acc=(base2 + d_recv * SUB, SUB, lambda s=s: zbuf[s, :, :]),
            )

        for t in range(3):
            d = lax.rem(z - t + 8, 4)
            slc = out_ref.at[pl.ds(base2 + d * SUB, SUB), :]
            step(slc, slc, z_right)

        for t in range(3):
            c = lax.rem(y - t + 8, 4)
            slc = out_ref.at[pl.ds(base + c * QTR, QTR), :]
            step(slc, slc, y_right)

        slc = out_ref.at[pl.ds(base, HALF), :]
        step(slc, slc, x_partner)

    return pl.pallas_call(
        body,
        out_shape=jax.ShapeDtypeStruct((M, N), jnp.float32),
        in_specs=[pl.BlockSpec(memory_space=pltpu.VMEM)],
        out_specs=pl.BlockSpec(memory_space=pltpu.VMEM),
        scratch_shapes=[
            pltpu.VMEM((HALF, N), jnp.float32),
            pltpu.VMEM((3, QTR, N), jnp.float32),
            pltpu.VMEM((3, SUB, N), jnp.float32),
            pltpu.SemaphoreType.DMA,
            pltpu.SemaphoreType.DMA((14,)),
        ],
    )(x)
